# baseline (speedup 1.0000x reference)
import functools

import jax
import jax.numpy as jnp
from jax import lax
from jax.experimental import pallas as pl
from jax.experimental.pallas import tpu as pltpu
from jax.experimental.pallas import tpu_sc as plsc

N_NODES = 100000
N_EDGES = 1600000

NC, NS = 2, 16
NW = NC * NS
GROUP = 128
CH = 16
ROWS_CH = CH * GROUP

NT = 102400
E_PAD = NW * 400 * GROUP
ER = E_PAD // 8

D = 16


def _sc_mesh():
    return plsc.VectorSubcoreMesh(
        core_axis_name="c", subcore_axis_name="s", num_cores=NC, num_subcores=NS)


GATHER_SPLIT = (640, 160)


def _gather_pallas(table, src2, dst2, eattr=None):
    with_e = eattr is not None
    out_type = [jax.ShapeDtypeStruct((E_PAD, D), jnp.float32),
                jax.ShapeDtypeStruct((E_PAD, D), jnp.float32)]
    scratch = [pltpu.VMEM((CH, GROUP), jnp.int32),
               pltpu.VMEM((ROWS_CH, D), jnp.float32),
               pltpu.SemaphoreType.DMA]
    if with_e:
        out_type.append(jax.ShapeDtypeStruct((E_PAD, D), jnp.float32))
        scratch.append(pltpu.VMEM((ROWS_CH, D), jnp.float32))
    E_SLOTS_FULL = N_EDGES // ROWS_CH
    E_REM = N_EDGES - E_SLOTS_FULL * ROWS_CH
    SLOTS_PW = (E_PAD // ROWS_CH) // NW

    @functools.partial(
        pl.kernel,
        out_type=tuple(out_type),
        mesh=_sc_mesh(),
        scratch_types=scratch,
        compiler_params=pltpu.CompilerParams(use_tc_tiling_on_sc=False),
    )
    def k(*refs):
        if with_e:
            table_h, src_h, dst_h, ea_h, hs_h, hd_h, e0_h, idx_v, rows_v, sem, ebuf = refs
        else:
            table_h, src_h, dst_h, hs_h, hd_h, idx_v, rows_v, sem = refs
        c = lax.axis_index("c")
        s = lax.axis_index("s")
        wid = s * NC + c
        w0, w1 = GATHER_SPLIT
        base = jnp.where(c == 0, s * w0, NS * w0 + s * w1)
        n_chunks = jnp.where(c == 0, w0 // CH, w1 // CH)

        if with_e:
            def epad_body(i, carry):
                slot = wid * SLOTS_PW + i
                r0 = pl.multiple_of(slot * ROWS_CH, ROWS_CH)

                def full():
                    pltpu.sync_copy(ea_h.at[pl.ds(r0, ROWS_CH)],
                                    ebuf.at[:, pl.ds(0, 6)])
                    pltpu.sync_copy(ebuf, e0_h.at[pl.ds(r0, ROWS_CH)])

                def partial():
                    pltpu.sync_copy(ea_h.at[pl.ds(r0, E_REM)],
                                    ebuf.at[pl.ds(0, E_REM), pl.ds(0, 6)])
                    pltpu.sync_copy(ebuf.at[pl.ds(0, E_REM)],
                                    e0_h.at[pl.ds(r0, E_REM)])

                pl.when(slot < E_SLOTS_FULL)(full)
                pl.when(slot == E_SLOTS_FULL)(partial)
                return carry
            lax.fori_loop(0, SLOTS_PW, epad_body, 0)

        def side(idx2, out_h):
            def body(i, carry):
                g0 = pl.multiple_of(base + i * CH, CH)
                pltpu.sync_copy(idx2.at[pl.ds(g0, CH)], idx_v)
                copies = []
                for b in range(CH):
                    copies.append(pltpu.async_copy(
                        table_h.at[idx_v.at[b]],
                        rows_v.at[pl.ds(b * GROUP, GROUP)], sem))
                for cp in copies:
                    cp.wait()
                r0 = pl.multiple_of(g0 * GROUP, ROWS_CH)
                pltpu.sync_copy(rows_v, out_h.at[pl.ds(r0, ROWS_CH)])
                return carry
            lax.fori_loop(0, n_chunks, body, 0)

        side(src_h, hs_h)
        side(dst_h, hd_h)

    if with_e:
        return k(table, src2, dst2, eattr)
    return k(table, src2, dst2)


def _scatter_pallas(part0, part1, dst2, zeros_nt, pack_e=False):
    per_t = E_PAD // NS
    groups_pt = per_t // GROUP
    CH_S = 4
    ROWS_S = CH_S * GROUP
    n_chunks = groups_pt // CH_S
    rows_init = NT // NS

    out_type = [jax.ShapeDtypeStruct((NT, D), jnp.float32),
                jax.ShapeDtypeStruct((NT, D), jnp.float32)]
    if pack_e:
        out_type.append(jax.ShapeDtypeStruct((E_PAD, 3), jnp.float32))

    @functools.partial(
        pl.kernel,
        out_type=tuple(out_type),
        mesh=_sc_mesh(),
        scratch_types=[pltpu.VMEM_SHARED((NT, D), jnp.float32),
                       pltpu.VMEM((CH_S, GROUP), jnp.int32),
                       pltpu.VMEM((ROWS_S, D), jnp.float32),
                       pltpu.SemaphoreType.DMA],
        compiler_params=pltpu.CompilerParams(use_tc_tiling_on_sc=False),
    )
    def k(*refs):
        if pack_e:
            (p0_h, p1_h, dst_h, z_h, a0_h, a1_h, ep_h,
             acc_s, idx_v, rows_v, sem) = refs
        else:
            p0_h, p1_h, dst_h, z_h, a0_h, a1_h, acc_s, idx_v, rows_v, sem = refs
            ep_h = None
        c = lax.axis_index("c")
        s = lax.axis_index("s")
        r_init = pl.multiple_of(s * rows_init, rows_init)
        pltpu.sync_copy(z_h.at[pl.ds(r_init, rows_init)],
                        acc_s.at[pl.ds(r_init, rows_init)])
        plsc.subcore_barrier()

        def scan_part(part_h, pack):
            def body(i, carry):
                g0 = pl.multiple_of(s * groups_pt + i * CH_S, CH_S)
                pltpu.sync_copy(dst_h.at[pl.ds(g0, CH_S)], idx_v)
                r0 = pl.multiple_of(s * per_t + i * ROWS_S, ROWS_S)
                pltpu.sync_copy(part_h.at[pl.ds(r0, ROWS_S)], rows_v)
                for b in range(CH_S):
                    pltpu.sync_copy(rows_v.at[pl.ds(b * GROUP, GROUP)],
                                    acc_s.at[idx_v.at[b]], add=True)
                if pack:
                    pltpu.sync_copy(rows_v.at[:, pl.ds(0, 3)],
                                    ep_h.at[pl.ds(r0, ROWS_S)])
                return carry
            lax.fori_loop(0, n_chunks, body, 0)

        pl.when(c == 0)(lambda: scan_part(p0_h, False))
        pl.when(c == 1)(lambda: scan_part(p1_h, pack_e))
        plsc.subcore_barrier()

        def writeout(out_h):
            pltpu.sync_copy(acc_s.at[pl.ds(r_init, rows_init)],
                            out_h.at[pl.ds(r_init, rows_init)])

        pl.when(c == 0)(lambda: writeout(a0_h))
        pl.when(c == 1)(lambda: writeout(a1_h))

    return k(part0, part1, dst2, zeros_nt)


def _edge_mlp_pallas(hs, hd, e_in, wa, we, wc, b1, w2, b2, w3=None, b3=None):
    BR = 1024
    grid = ER // BR
    n3 = 0 if w3 is None else 2

    def body(hs_r, hd_r, e_r, wa_r, we_r, wc_r, b1_r, w2_r, b2_r, *rest):
        out_r = rest[-1]
        h1 = (jnp.dot(hs_r[...], wa_r[...], preferred_element_type=jnp.float32)
              + jnp.dot(e_r[...], we_r[...], preferred_element_type=jnp.float32)
              + jnp.dot(hd_r[...], wc_r[...], preferred_element_type=jnp.float32)
              + b1_r[...])
        h1 = jnp.maximum(h1, 0.0)
        h2 = jnp.dot(h1, w2_r[...], preferred_element_type=jnp.float32) + b2_r[...]
        h2 = jnp.maximum(h2, 0.0)
        if n3:
            w3_r, b3_r = rest[0], rest[1]
            out_r[...] = jnp.dot(h2, w3_r[...],
                                 preferred_element_type=jnp.float32) + b3_r[...]
        else:
            out_r[...] = h2

    row_spec = pl.BlockSpec((BR, 128), lambda i: (i, 0))
    full_spec = lambda a: pl.BlockSpec(a.shape, lambda i: (0, 0))
    ins = [hs, hd, e_in, wa, we, wc, b1, w2, b2]
    if n3:
        ins += [w3, b3]
    in_specs = [row_spec] * 3 + [full_spec(a) for a in ins[3:]]
    return pl.pallas_call(
        body,
        grid=(grid,),
        in_specs=in_specs,
        out_specs=row_spec,
        out_shape=jax.ShapeDtypeStruct((ER, 128), jnp.float32),
    )(*ins)


def _node_mlp_pallas(a0, a1, wa, wb, b1, w2, b2, w3=None, b3=None):
    NR = NT // 8
    BR = 512
    grid = NR // BR
    n3 = 0 if w3 is None else 2

    def body(a0_r, a1_r, wa_r, wb_r, b1_r, w2_r, b2_r, *rest):
        out_r = rest[-1]
        h1 = (jnp.dot(a0_r[...], wa_r[...], preferred_element_type=jnp.float32)
              + jnp.dot(a1_r[...], wb_r[...], preferred_element_type=jnp.float32)
              + b1_r[...])
        h1 = jnp.maximum(h1, 0.0)
        h2 = jnp.dot(h1, w2_r[...], preferred_element_type=jnp.float32) + b2_r[...]
        h2 = jnp.maximum(h2, 0.0)
        if n3:
            w3_r, b3_r = rest[0], rest[1]
            out_r[...] = jnp.dot(h2, w3_r[...],
                                 preferred_element_type=jnp.float32) + b3_r[...]
        else:
            out_r[...] = h2

    row_spec = pl.BlockSpec((BR, 128), lambda i: (i, 0))
    full_spec = lambda a: pl.BlockSpec(a.shape, lambda i: (0, 0))
    ins = [a0, a1, wa, wb, b1, w2, b2]
    if n3:
        ins += [w3, b3]
    in_specs = [row_spec] * 2 + [full_spec(a) for a in ins[2:]]
    return pl.pallas_call(
        body,
        grid=(grid,),
        in_specs=in_specs,
        out_specs=row_spec,
        out_shape=jax.ShapeDtypeStruct((NR, 128), jnp.float32),
    )(*ins)


_EYE8 = None


def _blk(w):
    return jnp.kron(jnp.eye(8, dtype=w.dtype), w)


def _pad_rows(w, rows):
    return jnp.pad(w, ((0, rows - w.shape[0]), (0, 0)))


def _pad_cols(w, cols):
    return jnp.pad(w, ((0, 0), (0, cols - w.shape[1])))


def _brow(b):
    b = jnp.pad(b, (0, D - b.shape[0]))
    return jnp.tile(b, 8).reshape(1, 128)


def kernel(x, edge_attr, edge_index, params):
    src = edge_index[0]
    dst = edge_index[1]
    n_pad_e = E_PAD - N_EDGES
    fill = jnp.full((n_pad_e,), N_NODES, dtype=jnp.int32)
    src2 = jnp.concatenate([src, fill]).reshape(E_PAD // GROUP, GROUP)
    dst2 = jnp.concatenate([dst, fill]).reshape(E_PAD // GROUP, GROUP)

    table = jnp.zeros((NT, D), jnp.float32).at[:N_NODES, :x.shape[1]].set(x)
    zeros_nt = jnp.zeros((NT, D), jnp.float32)

    edge_attr = jnp.asarray(edge_attr)

    def edge_w(name, hs_w, e_w):
        (w1, b1), (w2, b2), *tail = params[name]
        wa = _blk(_pad_rows(w1[:hs_w], D))
        we = _blk(_pad_rows(w1[hs_w:hs_w + e_w], D))
        wc = _blk(_pad_rows(w1[hs_w + e_w:], D))
        out = [wa, we, wc, _brow(b1), _blk(w2), _brow(b2)]
        if tail:
            (w3, b3), = tail
            out += [_blk(_pad_cols(w3, D)), _brow(b3)]
        return out

    def node_w(name, hs_w):
        (w1, b1), (w2, b2), *tail = params[name]
        wa = _blk(_pad_rows(w1[:hs_w], D))
        wb = _blk(_pad_rows(w1[hs_w:], D))
        out = [wa, wb, _brow(b1), _blk(w2), _brow(b2)]
        if tail:
            (w3, b3), = tail
            out += [_blk(_pad_cols(w3, D)), _brow(b3)]
        return out

    h_widths = {"1": 12, "2": 16, "3": 16}
    e_widths = {"1": 6, "2": 16, "3": 16}

    e = None
    for li in ("1", "2", "3"):
        if li == "1":
            hs, hd, e0 = _gather_pallas(table, src2, dst2, edge_attr)
            e = e0.reshape(ER, 128)
        else:
            hs, hd = _gather_pallas(table, src2, dst2)
        e = _edge_mlp_pallas(hs.reshape(ER, 128), hd.reshape(ER, 128), e,
                             *edge_w("e" + li, h_widths[li], e_widths[li]))
        if li == "3":
            a0, a1, e_pack = _scatter_pallas(hs, e.reshape(E_PAD, D), dst2,
                                             zeros_nt, pack_e=True)
        else:
            a0, a1 = _scatter_pallas(hs, e.reshape(E_PAD, D), dst2, zeros_nt)
        table8 = _node_mlp_pallas(a0.reshape(NT // 8, 128),
                                  a1.reshape(NT // 8, 128),
                                  *node_w("n" + li, h_widths[li]))
        table = table8.reshape(NT, D)

    h_out = table[:N_NODES, :3]
    e_out = e_pack[:N_EDGES]
    return (h_out, e_out)


# speedup vs baseline: 1.4067x; 1.4067x over previous
import functools

import jax
import jax.numpy as jnp
from jax import lax
from jax.experimental import pallas as pl
from jax.experimental.pallas import tpu as pltpu
from jax.experimental.pallas import tpu_sc as plsc

N_NODES = 100000
N_EDGES = 1600000

NC, NS = 2, 16
NW = NC * NS
GROUP = 128
CH = 16
ROWS_CH = CH * GROUP

NT = 102400
E_PAD = NW * 400 * GROUP
ER = E_PAD // 8

D = 16


def _sc_mesh():
    return plsc.VectorSubcoreMesh(
        core_axis_name="c", subcore_axis_name="s", num_cores=NC, num_subcores=NS)


GATHER_SPLIT = (640, 160)


def _gather_pallas(table, src2, dst2, eattr=None):
    with_e = eattr is not None
    out_type = [jax.ShapeDtypeStruct((E_PAD, D), jnp.float32),
                jax.ShapeDtypeStruct((E_PAD, D), jnp.float32)]
    scratch = [pltpu.VMEM((CH, GROUP), jnp.int32),
               pltpu.VMEM((ROWS_CH, D), jnp.float32),
               pltpu.SemaphoreType.DMA]
    if with_e:
        out_type.append(jax.ShapeDtypeStruct((E_PAD, D), jnp.float32))
        scratch.append(pltpu.VMEM((ROWS_CH, D), jnp.float32))
    E_SLOTS_FULL = N_EDGES // ROWS_CH
    E_REM = N_EDGES - E_SLOTS_FULL * ROWS_CH
    SLOTS_PW = (E_PAD // ROWS_CH) // NW

    @functools.partial(
        pl.kernel,
        out_type=tuple(out_type),
        mesh=_sc_mesh(),
        scratch_types=scratch,
        compiler_params=pltpu.CompilerParams(use_tc_tiling_on_sc=False),
    )
    def k(*refs):
        if with_e:
            table_h, src_h, dst_h, ea_h, hs_h, hd_h, e0_h, idx_v, rows_v, sem, ebuf = refs
        else:
            table_h, src_h, dst_h, hs_h, hd_h, idx_v, rows_v, sem = refs
        c = lax.axis_index("c")
        s = lax.axis_index("s")
        wid = s * NC + c
        w0, w1 = GATHER_SPLIT
        base = jnp.where(c == 0, s * w0, NS * w0 + s * w1)
        n_chunks = jnp.where(c == 0, w0 // CH, w1 // CH)

        if with_e:
            def epad_body(i, carry):
                slot = wid * SLOTS_PW + i
                r0 = pl.multiple_of(slot * ROWS_CH, ROWS_CH)

                def full():
                    pltpu.sync_copy(ea_h.at[pl.ds(r0, ROWS_CH)],
                                    ebuf.at[:, pl.ds(0, 8)])
                    pltpu.sync_copy(ebuf, e0_h.at[pl.ds(r0, ROWS_CH)])

                def partial():
                    pltpu.sync_copy(ea_h.at[pl.ds(r0, E_REM)],
                                    ebuf.at[pl.ds(0, E_REM), pl.ds(0, 8)])
                    pltpu.sync_copy(ebuf.at[pl.ds(0, E_REM)],
                                    e0_h.at[pl.ds(r0, E_REM)])

                pl.when(slot < E_SLOTS_FULL)(full)
                pl.when(slot == E_SLOTS_FULL)(partial)
                return carry
            lax.fori_loop(0, SLOTS_PW, epad_body, 0)

        def side(idx2, out_h):
            def body(i, carry):
                g0 = pl.multiple_of(base + i * CH, CH)
                pltpu.sync_copy(idx2.at[pl.ds(g0, CH)], idx_v)
                copies = []
                for b in range(CH):
                    copies.append(pltpu.async_copy(
                        table_h.at[idx_v.at[b]],
                        rows_v.at[pl.ds(b * GROUP, GROUP)], sem))
                for cp in copies:
                    cp.wait()
                r0 = pl.multiple_of(g0 * GROUP, ROWS_CH)
                pltpu.sync_copy(rows_v, out_h.at[pl.ds(r0, ROWS_CH)])
                return carry
            lax.fori_loop(0, n_chunks, body, 0)

        side(src_h, hs_h)
        side(dst_h, hd_h)

    if with_e:
        return k(table, src2, dst2, eattr)
    return k(table, src2, dst2)


def _scatter_pallas(part0, part1, dst2, zeros_nt, pack_e=False):
    per_t = E_PAD // NS
    groups_pt = per_t // GROUP
    CH_S = 4
    ROWS_S = CH_S * GROUP
    n_chunks = groups_pt // CH_S
    rows_init = NT // NS

    out_type = [jax.ShapeDtypeStruct((NT, D), jnp.float32),
                jax.ShapeDtypeStruct((NT, D), jnp.float32)]
    scratch = [pltpu.VMEM_SHARED((NT, D), jnp.float32),
               pltpu.VMEM((CH_S, GROUP), jnp.int32),
               pltpu.VMEM((ROWS_S, D), jnp.float32),
               pltpu.SemaphoreType.DMA]
    if pack_e:
        out_type.append(jax.ShapeDtypeStruct((E_PAD * 3,), jnp.float32))
        scratch.append(pltpu.VMEM((ROWS_S * 3,), jnp.float32))

    @functools.partial(
        pl.kernel,
        out_type=tuple(out_type),
        mesh=_sc_mesh(),
        scratch_types=scratch,
        compiler_params=pltpu.CompilerParams(
            use_tc_tiling_on_sc=False,
            needs_layout_passes=not pack_e),
    )
    def k(*refs):
        if pack_e:
            (p0_h, p1_h, dst_h, z_h, a0_h, a1_h, ep_h,
             acc_s, idx_v, rows_v, sem, pk_v) = refs
        else:
            p0_h, p1_h, dst_h, z_h, a0_h, a1_h, acc_s, idx_v, rows_v, sem = refs
            ep_h = pk_v = None
        c = lax.axis_index("c")
        s = lax.axis_index("s")
        r_init = pl.multiple_of(s * rows_init, rows_init)
        pltpu.sync_copy(z_h.at[pl.ds(r_init, rows_init)],
                        acc_s.at[pl.ds(r_init, rows_init)])
        plsc.subcore_barrier()

        def scan_part(part_h, pack):
            def body(i, carry):
                g0 = pl.multiple_of(s * groups_pt + i * CH_S, CH_S)
                pltpu.sync_copy(dst_h.at[pl.ds(g0, CH_S)], idx_v)
                r0 = pl.multiple_of(s * per_t + i * ROWS_S, ROWS_S)
                pltpu.sync_copy(part_h.at[pl.ds(r0, ROWS_S)], rows_v)
                for b in range(CH_S):
                    pltpu.sync_copy(rows_v.at[pl.ds(b * GROUP, GROUP)],
                                    acc_s.at[idx_v.at[b]], add=True)
                if pack:
                    iot = lax.iota(jnp.int32, 16)

                    def pk_body(j, carry2):
                        e0 = j * 16
                        ev = e0 + iot
                        for cc in range(3):
                            g = plsc.load_gather(
                                rows_v, [ev, jnp.full((16,), cc, jnp.int32)])
                            plsc.store_scatter(pk_v, [ev * 3 + cc], g)
                        return carry2
                    lax.fori_loop(0, ROWS_S // 16, pk_body, 0)
                    r3 = pl.multiple_of(r0 * 3, ROWS_S * 3)
                    pltpu.sync_copy(pk_v, ep_h.at[pl.ds(r3, ROWS_S * 3)])
                return carry
            lax.fori_loop(0, n_chunks, body, 0)

        pl.when(c == 0)(lambda: scan_part(p0_h, False))
        pl.when(c == 1)(lambda: scan_part(p1_h, pack_e))
        plsc.subcore_barrier()

        def writeout(out_h):
            pltpu.sync_copy(acc_s.at[pl.ds(r_init, rows_init)],
                            out_h.at[pl.ds(r_init, rows_init)])

        pl.when(c == 0)(lambda: writeout(a0_h))
        pl.when(c == 1)(lambda: writeout(a1_h))

    return k(part0, part1, dst2, zeros_nt)


def _edge_mlp_pallas(hs, hd, e_in, wa, we, wc, b1, w2, b2, w3=None, b3=None):
    BR = 1024
    grid = ER // BR
    n3 = 0 if w3 is None else 2

    def body(hs_r, hd_r, e_r, wa_r, we_r, wc_r, b1_r, w2_r, b2_r, *rest):
        out_r = rest[-1]
        h1 = (jnp.dot(hs_r[...], wa_r[...], preferred_element_type=jnp.float32)
              + jnp.dot(e_r[...], we_r[...], preferred_element_type=jnp.float32)
              + jnp.dot(hd_r[...], wc_r[...], preferred_element_type=jnp.float32)
              + b1_r[...])
        h1 = jnp.maximum(h1, 0.0)
        h2 = jnp.dot(h1, w2_r[...], preferred_element_type=jnp.float32) + b2_r[...]
        h2 = jnp.maximum(h2, 0.0)
        if n3:
            w3_r, b3_r = rest[0], rest[1]
            out_r[...] = jnp.dot(h2, w3_r[...],
                                 preferred_element_type=jnp.float32) + b3_r[...]
        else:
            out_r[...] = h2

    row_spec = pl.BlockSpec((BR, 128), lambda i: (i, 0))
    full_spec = lambda a: pl.BlockSpec(a.shape, lambda i: (0, 0))
    ins = [hs, hd, e_in, wa, we, wc, b1, w2, b2]
    if n3:
        ins += [w3, b3]
    in_specs = [row_spec] * 3 + [full_spec(a) for a in ins[3:]]
    return pl.pallas_call(
        body,
        grid=(grid,),
        in_specs=in_specs,
        out_specs=row_spec,
        out_shape=jax.ShapeDtypeStruct((ER, 128), jnp.float32),
    )(*ins)


def _node_mlp_pallas(a0, a1, wa, wb, b1, w2, b2, w3=None, b3=None):
    NR = NT // 8
    BR = 512
    grid = NR // BR
    n3 = 0 if w3 is None else 2

    def body(a0_r, a1_r, wa_r, wb_r, b1_r, w2_r, b2_r, *rest):
        out_r = rest[-1]
        h1 = (jnp.dot(a0_r[...], wa_r[...], preferred_element_type=jnp.float32)
              + jnp.dot(a1_r[...], wb_r[...], preferred_element_type=jnp.float32)
              + b1_r[...])
        h1 = jnp.maximum(h1, 0.0)
        h2 = jnp.dot(h1, w2_r[...], preferred_element_type=jnp.float32) + b2_r[...]
        h2 = jnp.maximum(h2, 0.0)
        if n3:
            w3_r, b3_r = rest[0], rest[1]
            out_r[...] = jnp.dot(h2, w3_r[...],
                                 preferred_element_type=jnp.float32) + b3_r[...]
        else:
            out_r[...] = h2

    row_spec = pl.BlockSpec((BR, 128), lambda i: (i, 0))
    full_spec = lambda a: pl.BlockSpec(a.shape, lambda i: (0, 0))
    ins = [a0, a1, wa, wb, b1, w2, b2]
    if n3:
        ins += [w3, b3]
    in_specs = [row_spec] * 2 + [full_spec(a) for a in ins[2:]]
    return pl.pallas_call(
        body,
        grid=(grid,),
        in_specs=in_specs,
        out_specs=row_spec,
        out_shape=jax.ShapeDtypeStruct((NR, 128), jnp.float32),
    )(*ins)


_EYE8 = None


def _blk(w):
    return jnp.kron(jnp.eye(8, dtype=w.dtype), w)


def _pad_rows(w, rows):
    return jnp.pad(w, ((0, rows - w.shape[0]), (0, 0)))


def _pad_cols(w, cols):
    return jnp.pad(w, ((0, 0), (0, cols - w.shape[1])))


def _brow(b):
    b = jnp.pad(b, (0, D - b.shape[0]))
    return jnp.tile(b, 8).reshape(1, 128)


def kernel(x, edge_attr, edge_index, params):
    src = edge_index[0]
    dst = edge_index[1]
    n_pad_e = E_PAD - N_EDGES
    fill = jnp.full((n_pad_e,), N_NODES, dtype=jnp.int32)
    src2 = jnp.concatenate([src, fill]).reshape(E_PAD // GROUP, GROUP)
    dst2 = jnp.concatenate([dst, fill]).reshape(E_PAD // GROUP, GROUP)

    table = jnp.zeros((NT, D), jnp.float32).at[:N_NODES, :x.shape[1]].set(x)
    zeros_nt = jnp.zeros((NT, D), jnp.float32)

    edge_attr8 = jnp.pad(edge_attr, ((0, 0), (0, 2)))

    def edge_w(name, hs_w, e_w):
        (w1, b1), (w2, b2), *tail = params[name]
        wa = _blk(_pad_rows(w1[:hs_w], D))
        we = _blk(_pad_rows(w1[hs_w:hs_w + e_w], D))
        wc = _blk(_pad_rows(w1[hs_w + e_w:], D))
        out = [wa, we, wc, _brow(b1), _blk(w2), _brow(b2)]
        if tail:
            (w3, b3), = tail
            out += [_blk(_pad_cols(w3, D)), _brow(b3)]
        return out

    def node_w(name, hs_w):
        (w1, b1), (w2, b2), *tail = params[name]
        wa = _blk(_pad_rows(w1[:hs_w], D))
        wb = _blk(_pad_rows(w1[hs_w:], D))
        out = [wa, wb, _brow(b1), _blk(w2), _brow(b2)]
        if tail:
            (w3, b3), = tail
            out += [_blk(_pad_cols(w3, D)), _brow(b3)]
        return out

    h_widths = {"1": 12, "2": 16, "3": 16}
    e_widths = {"1": 6, "2": 16, "3": 16}

    e = None
    for li in ("1", "2", "3"):
        if li == "1":
            hs, hd, e0 = _gather_pallas(table, src2, dst2, edge_attr8)
            e = e0.reshape(ER, 128)
        else:
            hs, hd = _gather_pallas(table, src2, dst2)
        e = _edge_mlp_pallas(hs.reshape(ER, 128), hd.reshape(ER, 128), e,
                             *edge_w("e" + li, h_widths[li], e_widths[li]))
        if li == "3":
            a0, a1, e_pack = _scatter_pallas(hs, e.reshape(E_PAD, D), dst2,
                                             zeros_nt, pack_e=True)
        else:
            a0, a1 = _scatter_pallas(hs, e.reshape(E_PAD, D), dst2, zeros_nt)
        table8 = _node_mlp_pallas(a0.reshape(NT // 8, 128),
                                  a1.reshape(NT // 8, 128),
                                  *node_w("n" + li, h_widths[li]))
        table = table8.reshape(NT, D)

    h_out = table[:N_NODES, :3]
    e_out = e_pack.reshape(E_PAD, 3)[:N_EDGES]
    return (h_out, e_out)


# speedup vs baseline: 1.5949x; 1.1338x over previous
import functools

import jax
import jax.numpy as jnp
from jax import lax
from jax.experimental import pallas as pl
from jax.experimental.pallas import tpu as pltpu
from jax.experimental.pallas import tpu_sc as plsc

N_NODES = 100000
N_EDGES = 1600000

NC, NS = 2, 16
NW = NC * NS
GROUP = 128
CH = 16
ROWS_CH = CH * GROUP

NT = 102400
E_PAD = NW * 400 * GROUP
ER = E_PAD // 8

D = 16


def _sc_mesh():
    return plsc.VectorSubcoreMesh(
        core_axis_name="c", subcore_axis_name="s", num_cores=NC, num_subcores=NS)


GATHER_SPLIT = (640, 160)


def _gather_pallas(table, src2, dst2):
    @functools.partial(
        pl.kernel,
        out_type=(jax.ShapeDtypeStruct((E_PAD, D), jnp.float32),
                  jax.ShapeDtypeStruct((E_PAD, D), jnp.float32)),
        mesh=_sc_mesh(),
        scratch_types=[pltpu.VMEM((CH, GROUP), jnp.int32),
                       pltpu.VMEM((ROWS_CH, D), jnp.float32),
                       pltpu.SemaphoreType.DMA],
        compiler_params=pltpu.CompilerParams(use_tc_tiling_on_sc=False),
    )
    def k(table_h, src_h, dst_h, hs_h, hd_h, idx_v, rows_v, sem):
        c = lax.axis_index("c")
        s = lax.axis_index("s")
        w0, w1 = GATHER_SPLIT
        base = jnp.where(c == 0, s * w0, NS * w0 + s * w1)
        n_chunks = jnp.where(c == 0, w0 // CH, w1 // CH)

        def side(idx2, out_h):
            def body(i, carry):
                g0 = pl.multiple_of(base + i * CH, CH)
                pltpu.sync_copy(idx2.at[pl.ds(g0, CH)], idx_v)
                copies = []
                for b in range(CH):
                    copies.append(pltpu.async_copy(
                        table_h.at[idx_v.at[b]],
                        rows_v.at[pl.ds(b * GROUP, GROUP)], sem))
                for cp in copies:
                    cp.wait()
                r0 = pl.multiple_of(g0 * GROUP, ROWS_CH)
                pltpu.sync_copy(rows_v, out_h.at[pl.ds(r0, ROWS_CH)])
                return carry
            lax.fori_loop(0, n_chunks, body, 0)

        side(src_h, hs_h)
        side(dst_h, hd_h)

    return k(table, src2, dst2)


def _scatter_pallas(part0, part1, dst2, zeros_nt, pack_e=False):
    per_t = E_PAD // NS
    groups_pt = per_t // GROUP
    CH_S = 4
    ROWS_S = CH_S * GROUP
    n_chunks = groups_pt // CH_S
    rows_init = NT // NS

    out_type = [jax.ShapeDtypeStruct((NT, D), jnp.float32),
                jax.ShapeDtypeStruct((NT, D), jnp.float32)]
    scratch = [pltpu.VMEM_SHARED((NT, D), jnp.float32),
               pltpu.VMEM((CH_S, GROUP), jnp.int32),
               pltpu.VMEM((ROWS_S, D), jnp.float32),
               pltpu.SemaphoreType.DMA]
    if pack_e:
        out_type.append(jax.ShapeDtypeStruct((E_PAD * 3,), jnp.float32))
        scratch.append(pltpu.VMEM((ROWS_S * 3,), jnp.float32))

    @functools.partial(
        pl.kernel,
        out_type=tuple(out_type),
        mesh=_sc_mesh(),
        scratch_types=scratch,
        compiler_params=pltpu.CompilerParams(
            use_tc_tiling_on_sc=False,
            needs_layout_passes=not pack_e),
    )
    def k(*refs):
        if pack_e:
            (p0_h, p1_h, dst_h, z_h, a0_h, a1_h, ep_h,
             acc_s, idx_v, rows_v, sem, pk_v) = refs
        else:
            p0_h, p1_h, dst_h, z_h, a0_h, a1_h, acc_s, idx_v, rows_v, sem = refs
            ep_h = pk_v = None
        c = lax.axis_index("c")
        s = lax.axis_index("s")
        r_init = pl.multiple_of(s * rows_init, rows_init)
        pltpu.sync_copy(z_h.at[pl.ds(r_init, rows_init)],
                        acc_s.at[pl.ds(r_init, rows_init)])
        plsc.subcore_barrier()

        def scan_part(part_h, pack):
            def body(i, carry):
                g0 = pl.multiple_of(s * groups_pt + i * CH_S, CH_S)
                pltpu.sync_copy(dst_h.at[pl.ds(g0, CH_S)], idx_v)
                r0 = pl.multiple_of(s * per_t + i * ROWS_S, ROWS_S)
                pltpu.sync_copy(part_h.at[pl.ds(r0, ROWS_S)], rows_v)
                for b in range(CH_S):
                    pltpu.sync_copy(rows_v.at[pl.ds(b * GROUP, GROUP)],
                                    acc_s.at[idx_v.at[b]], add=True)
                if pack:
                    iot = lax.iota(jnp.int32, 16)

                    def pk_body(j, carry2):
                        e0 = j * 16
                        ev = e0 + iot
                        for cc in range(3):
                            g = plsc.load_gather(
                                rows_v, [ev, jnp.full((16,), cc, jnp.int32)])
                            plsc.store_scatter(pk_v, [ev * 3 + cc], g)
                        return carry2
                    lax.fori_loop(0, ROWS_S // 16, pk_body, 0)
                    r3 = pl.multiple_of(r0 * 3, ROWS_S * 3)
                    pltpu.sync_copy(pk_v, ep_h.at[pl.ds(r3, ROWS_S * 3)])
                return carry
            lax.fori_loop(0, n_chunks, body, 0)

        pl.when(c == 0)(lambda: scan_part(p0_h, False))
        pl.when(c == 1)(lambda: scan_part(p1_h, pack_e))
        plsc.subcore_barrier()

        def writeout(out_h):
            pltpu.sync_copy(acc_s.at[pl.ds(r_init, rows_init)],
                            out_h.at[pl.ds(r_init, rows_init)])

        pl.when(c == 0)(lambda: writeout(a0_h))
        pl.when(c == 1)(lambda: writeout(a1_h))

    return k(part0, part1, dst2, zeros_nt)


def _edge_mlp_pallas(hs, hd, e_in, wa, we, wc, b1, w2, b2, w3=None, b3=None):
    BR = 1024
    grid = ER // BR
    n3 = 0 if w3 is None else 2

    def body(hs_r, hd_r, e_r, wa_r, we_r, wc_r, b1_r, w2_r, b2_r, *rest):
        out_r = rest[-1]
        h1 = (jnp.dot(hs_r[...], wa_r[...], preferred_element_type=jnp.float32)
              + jnp.dot(e_r[...], we_r[...], preferred_element_type=jnp.float32)
              + jnp.dot(hd_r[...], wc_r[...], preferred_element_type=jnp.float32)
              + b1_r[...])
        h1 = jnp.maximum(h1, 0.0)
        h2 = jnp.dot(h1, w2_r[...], preferred_element_type=jnp.float32) + b2_r[...]
        h2 = jnp.maximum(h2, 0.0)
        if n3:
            w3_r, b3_r = rest[0], rest[1]
            out_r[...] = jnp.dot(h2, w3_r[...],
                                 preferred_element_type=jnp.float32) + b3_r[...]
        else:
            out_r[...] = h2

    row_spec = pl.BlockSpec((BR, 128), lambda i: (i, 0))
    e_spec = pl.BlockSpec((BR, e_in.shape[1]), lambda i: (i, 0))
    full_spec = lambda a: pl.BlockSpec(a.shape, lambda i: (0, 0))
    ins = [hs, hd, e_in, wa, we, wc, b1, w2, b2]
    if n3:
        ins += [w3, b3]
    in_specs = [row_spec, row_spec, e_spec] + [full_spec(a) for a in ins[3:]]
    return pl.pallas_call(
        body,
        grid=(grid,),
        in_specs=in_specs,
        out_specs=row_spec,
        out_shape=jax.ShapeDtypeStruct((ER, 128), jnp.float32),
    )(*ins)


def _node_mlp_pallas(a0, a1, wa, wb, b1, w2, b2, w3=None, b3=None):
    NR = NT // 8
    BR = 512
    grid = NR // BR
    n3 = 0 if w3 is None else 2

    def body(a0_r, a1_r, wa_r, wb_r, b1_r, w2_r, b2_r, *rest):
        out_r = rest[-1]
        h1 = (jnp.dot(a0_r[...], wa_r[...], preferred_element_type=jnp.float32)
              + jnp.dot(a1_r[...], wb_r[...], preferred_element_type=jnp.float32)
              + b1_r[...])
        h1 = jnp.maximum(h1, 0.0)
        h2 = jnp.dot(h1, w2_r[...], preferred_element_type=jnp.float32) + b2_r[...]
        h2 = jnp.maximum(h2, 0.0)
        if n3:
            w3_r, b3_r = rest[0], rest[1]
            out_r[...] = jnp.dot(h2, w3_r[...],
                                 preferred_element_type=jnp.float32) + b3_r[...]
        else:
            out_r[...] = h2

    row_spec = pl.BlockSpec((BR, 128), lambda i: (i, 0))
    full_spec = lambda a: pl.BlockSpec(a.shape, lambda i: (0, 0))
    ins = [a0, a1, wa, wb, b1, w2, b2]
    if n3:
        ins += [w3, b3]
    in_specs = [row_spec] * 2 + [full_spec(a) for a in ins[2:]]
    return pl.pallas_call(
        body,
        grid=(grid,),
        in_specs=in_specs,
        out_specs=row_spec,
        out_shape=jax.ShapeDtypeStruct((NR, 128), jnp.float32),
    )(*ins)


_EYE8 = None


def _blk(w):
    return jnp.kron(jnp.eye(8, dtype=w.dtype), w)


def _pad_rows(w, rows):
    return jnp.pad(w, ((0, rows - w.shape[0]), (0, 0)))


def _pad_cols(w, cols):
    return jnp.pad(w, ((0, 0), (0, cols - w.shape[1])))


def _brow(b):
    b = jnp.pad(b, (0, D - b.shape[0]))
    return jnp.tile(b, 8).reshape(1, 128)


def kernel(x, edge_attr, edge_index, params):
    src = edge_index[0]
    dst = edge_index[1]
    n_pad_e = E_PAD - N_EDGES
    fill = jnp.full((n_pad_e,), N_NODES, dtype=jnp.int32)
    src2 = jnp.concatenate([src, fill]).reshape(E_PAD // GROUP, GROUP)
    dst2 = jnp.concatenate([dst, fill]).reshape(E_PAD // GROUP, GROUP)

    table = jnp.zeros((NT, D), jnp.float32).at[:N_NODES, :x.shape[1]].set(x)
    zeros_nt = jnp.zeros((NT, D), jnp.float32)

    e48 = jnp.pad(edge_attr.reshape(N_EDGES // 8, 48),
                  ((0, ER - N_EDGES // 8), (0, 0)))

    def edge_w(name, hs_w, e_w):
        (w1, b1), (w2, b2), *tail = params[name]
        wa = _blk(_pad_rows(w1[:hs_w], D))
        we = _blk(w1[hs_w:hs_w + e_w])
        wc = _blk(_pad_rows(w1[hs_w + e_w:], D))
        out = [wa, we, wc, _brow(b1), _blk(w2), _brow(b2)]
        if tail:
            (w3, b3), = tail
            out += [_blk(_pad_cols(w3, D)), _brow(b3)]
        return out

    def node_w(name, hs_w):
        (w1, b1), (w2, b2), *tail = params[name]
        wa = _blk(_pad_rows(w1[:hs_w], D))
        wb = _blk(_pad_rows(w1[hs_w:], D))
        out = [wa, wb, _brow(b1), _blk(w2), _brow(b2)]
        if tail:
            (w3, b3), = tail
            out += [_blk(_pad_cols(w3, D)), _brow(b3)]
        return out

    h_widths = {"1": 12, "2": 16, "3": 16}
    e_widths = {"1": 6, "2": 16, "3": 16}

    e = e48
    for li in ("1", "2", "3"):
        hs, hd = _gather_pallas(table, src2, dst2)
        e = _edge_mlp_pallas(hs.reshape(ER, 128), hd.reshape(ER, 128), e,
                             *edge_w("e" + li, h_widths[li], e_widths[li]))
        if li == "3":
            a0, a1, e_pack = _scatter_pallas(hs, e.reshape(E_PAD, D), dst2,
                                             zeros_nt, pack_e=True)
        else:
            a0, a1 = _scatter_pallas(hs, e.reshape(E_PAD, D), dst2, zeros_nt)
        table8 = _node_mlp_pallas(a0.reshape(NT // 8, 128),
                                  a1.reshape(NT // 8, 128),
                                  *node_w("n" + li, h_widths[li]))
        table = table8.reshape(NT, D)

    h_out = table[:N_NODES, :3]
    e_out = e_pack.reshape(E_PAD, 3)[:N_EDGES]
    return (h_out, e_out)


# speedup vs baseline: 1.7416x; 1.0920x over previous
import functools

import jax
import jax.numpy as jnp
from jax import lax
from jax.experimental import pallas as pl
from jax.experimental.pallas import tpu as pltpu
from jax.experimental.pallas import tpu_sc as plsc

N_NODES = 100000
N_EDGES = 1600000

NC, NS = 2, 16
NW = NC * NS
GROUP = 128
CH = 16
ROWS_CH = CH * GROUP

NT = 102400
E_PAD = NW * 400 * GROUP
ER = E_PAD // 8

D = 16


def _sc_mesh():
    return plsc.VectorSubcoreMesh(
        core_axis_name="c", subcore_axis_name="s", num_cores=NC, num_subcores=NS)


GATHER_SPLIT = (640, 160)


def _gather_pallas(table, src2, dst2):
    @functools.partial(
        pl.kernel,
        out_type=(jax.ShapeDtypeStruct((E_PAD, D), jnp.float32),
                  jax.ShapeDtypeStruct((E_PAD, D), jnp.float32)),
        mesh=_sc_mesh(),
        scratch_types=[pltpu.VMEM((CH, GROUP), jnp.int32),
                       pltpu.VMEM((ROWS_CH, D), jnp.float32),
                       pltpu.SemaphoreType.DMA],
        compiler_params=pltpu.CompilerParams(use_tc_tiling_on_sc=False),
    )
    def k(table_h, src_h, dst_h, hs_h, hd_h, idx_v, rows_v, sem):
        c = lax.axis_index("c")
        s = lax.axis_index("s")
        w0, w1 = GATHER_SPLIT
        base = jnp.where(c == 0, s * w0, NS * w0 + s * w1)
        n_chunks = jnp.where(c == 0, w0 // CH, w1 // CH)

        def side(idx2, out_h):
            def body(i, carry):
                g0 = pl.multiple_of(base + i * CH, CH)
                pltpu.sync_copy(idx2.at[pl.ds(g0, CH)], idx_v)
                copies = []
                for b in range(CH):
                    copies.append(pltpu.async_copy(
                        table_h.at[idx_v.at[b]],
                        rows_v.at[pl.ds(b * GROUP, GROUP)], sem))
                for cp in copies:
                    cp.wait()
                r0 = pl.multiple_of(g0 * GROUP, ROWS_CH)
                pltpu.sync_copy(rows_v, out_h.at[pl.ds(r0, ROWS_CH)])
                return carry
            lax.fori_loop(0, n_chunks, body, 0)

        side(src_h, hs_h)
        side(dst_h, hd_h)

    return k(table, src2, dst2)


def _scatter_pallas(part0, part1, dst2, zeros_nt, pack_e=False):
    per_t = E_PAD // NS
    groups_pt = per_t // GROUP
    CH_S = 4
    ROWS_S = CH_S * GROUP
    n_chunks = groups_pt // CH_S
    rows_init = NT // NS

    out_type = [jax.ShapeDtypeStruct((NT, D), jnp.float32),
                jax.ShapeDtypeStruct((NT, D), jnp.float32)]
    scratch = [pltpu.VMEM_SHARED((NT, D), jnp.float32),
               pltpu.VMEM((CH_S, GROUP), jnp.int32),
               pltpu.VMEM((ROWS_S, D), jnp.float32),
               pltpu.SemaphoreType.DMA]
    if pack_e:
        out_type.append(jax.ShapeDtypeStruct((E_PAD * 3,), jnp.float32))
        scratch.append(pltpu.VMEM((ROWS_S * 3,), jnp.float32))

    @functools.partial(
        pl.kernel,
        out_type=tuple(out_type),
        mesh=_sc_mesh(),
        scratch_types=scratch,
        compiler_params=pltpu.CompilerParams(
            use_tc_tiling_on_sc=False,
            needs_layout_passes=not pack_e),
    )
    def k(*refs):
        if pack_e:
            (p0_h, p1_h, dst_h, z_h, a0_h, a1_h, ep_h,
             acc_s, idx_v, rows_v, sem, pk_v) = refs
        else:
            p0_h, p1_h, dst_h, z_h, a0_h, a1_h, acc_s, idx_v, rows_v, sem = refs
            ep_h = pk_v = None
        c = lax.axis_index("c")
        s = lax.axis_index("s")
        r_init = pl.multiple_of(s * rows_init, rows_init)
        pltpu.sync_copy(z_h.at[pl.ds(r_init, rows_init)],
                        acc_s.at[pl.ds(r_init, rows_init)])
        plsc.subcore_barrier()

        def scan_part(part_h, pack):
            def body(i, carry):
                g0 = pl.multiple_of(s * groups_pt + i * CH_S, CH_S)
                pltpu.sync_copy(dst_h.at[pl.ds(g0, CH_S)], idx_v)
                r0 = pl.multiple_of(s * per_t + i * ROWS_S, ROWS_S)
                pltpu.sync_copy(part_h.at[pl.ds(r0, ROWS_S)], rows_v)
                for b in range(CH_S):
                    pltpu.sync_copy(rows_v.at[pl.ds(b * GROUP, GROUP)],
                                    acc_s.at[idx_v.at[b]], add=True)
                if pack:
                    iot = lax.iota(jnp.int32, 16)

                    def pk_body(j, carry2):
                        ev = j * 16 + iot
                        for cc in range(3):
                            g = plsc.load_gather(
                                rows_v, [ev, jnp.full((16,), cc, jnp.int32)])
                            plsc.store_scatter(pk_v, [cc * ROWS_S + ev], g)
                        return carry2
                    lax.fori_loop(0, ROWS_S // 16, pk_body, 0)
                    for cc in range(3):
                        pltpu.sync_copy(
                            pk_v.at[pl.ds(cc * ROWS_S, ROWS_S)],
                            ep_h.at[pl.ds(pl.multiple_of(cc * E_PAD + r0,
                                                         ROWS_S), ROWS_S)])
                return carry
            lax.fori_loop(0, n_chunks, body, 0)

        pl.when(c == 0)(lambda: scan_part(p0_h, False))
        pl.when(c == 1)(lambda: scan_part(p1_h, pack_e))
        plsc.subcore_barrier()

        def writeout(out_h):
            pltpu.sync_copy(acc_s.at[pl.ds(r_init, rows_init)],
                            out_h.at[pl.ds(r_init, rows_init)])

        pl.when(c == 0)(lambda: writeout(a0_h))
        pl.when(c == 1)(lambda: writeout(a1_h))

    return k(part0, part1, dst2, zeros_nt)


def _edge_mlp_pallas(hs, hd, e_in, wa, we, wc, b1, w2, b2, w3=None, b3=None):
    BR = 1024
    grid = ER // BR
    n3 = 0 if w3 is None else 2

    def body(hs_r, hd_r, e_r, wa_r, we_r, wc_r, b1_r, w2_r, b2_r, *rest):
        out_r = rest[-1]
        h1 = (jnp.dot(hs_r[...], wa_r[...], preferred_element_type=jnp.float32)
              + jnp.dot(e_r[...], we_r[...], preferred_element_type=jnp.float32)
              + jnp.dot(hd_r[...], wc_r[...], preferred_element_type=jnp.float32)
              + b1_r[...])
        h1 = jnp.maximum(h1, 0.0)
        h2 = jnp.dot(h1, w2_r[...], preferred_element_type=jnp.float32) + b2_r[...]
        h2 = jnp.maximum(h2, 0.0)
        if n3:
            w3_r, b3_r = rest[0], rest[1]
            out_r[...] = jnp.dot(h2, w3_r[...],
                                 preferred_element_type=jnp.float32) + b3_r[...]
        else:
            out_r[...] = h2

    row_spec = pl.BlockSpec((BR, 128), lambda i: (i, 0))
    e_spec = pl.BlockSpec((BR, e_in.shape[1]), lambda i: (i, 0))
    full_spec = lambda a: pl.BlockSpec(a.shape, lambda i: (0, 0))
    ins = [hs, hd, e_in, wa, we, wc, b1, w2, b2]
    if n3:
        ins += [w3, b3]
    in_specs = [row_spec, row_spec, e_spec] + [full_spec(a) for a in ins[3:]]
    return pl.pallas_call(
        body,
        grid=(grid,),
        in_specs=in_specs,
        out_specs=row_spec,
        out_shape=jax.ShapeDtypeStruct((ER, 128), jnp.float32),
    )(*ins)


def _node_mlp_pallas(a0, a1, wa, wb, b1, w2, b2, w3=None, b3=None):
    NR = NT // 8
    BR = 512
    grid = NR // BR
    n3 = 0 if w3 is None else 2

    def body(a0_r, a1_r, wa_r, wb_r, b1_r, w2_r, b2_r, *rest):
        out_r = rest[-1]
        h1 = (jnp.dot(a0_r[...], wa_r[...], preferred_element_type=jnp.float32)
              + jnp.dot(a1_r[...], wb_r[...], preferred_element_type=jnp.float32)
              + b1_r[...])
        h1 = jnp.maximum(h1, 0.0)
        h2 = jnp.dot(h1, w2_r[...], preferred_element_type=jnp.float32) + b2_r[...]
        h2 = jnp.maximum(h2, 0.0)
        if n3:
            w3_r, b3_r = rest[0], rest[1]
            out_r[...] = jnp.dot(h2, w3_r[...],
                                 preferred_element_type=jnp.float32) + b3_r[...]
        else:
            out_r[...] = h2

    row_spec = pl.BlockSpec((BR, 128), lambda i: (i, 0))
    full_spec = lambda a: pl.BlockSpec(a.shape, lambda i: (0, 0))
    ins = [a0, a1, wa, wb, b1, w2, b2]
    if n3:
        ins += [w3, b3]
    in_specs = [row_spec] * 2 + [full_spec(a) for a in ins[2:]]
    return pl.pallas_call(
        body,
        grid=(grid,),
        in_specs=in_specs,
        out_specs=row_spec,
        out_shape=jax.ShapeDtypeStruct((NR, 128), jnp.float32),
    )(*ins)


_EYE8 = None


def _blk(w):
    return jnp.kron(jnp.eye(8, dtype=w.dtype), w)


def _pad_rows(w, rows):
    return jnp.pad(w, ((0, rows - w.shape[0]), (0, 0)))


def _pad_cols(w, cols):
    return jnp.pad(w, ((0, 0), (0, cols - w.shape[1])))


def _brow(b):
    b = jnp.pad(b, (0, D - b.shape[0]))
    return jnp.tile(b, 8).reshape(1, 128)


def kernel(x, edge_attr, edge_index, params):
    src = edge_index[0]
    dst = edge_index[1]
    n_pad_e = E_PAD - N_EDGES
    fill = jnp.full((n_pad_e,), N_NODES, dtype=jnp.int32)
    src2 = jnp.concatenate([src, fill]).reshape(E_PAD // GROUP, GROUP)
    dst2 = jnp.concatenate([dst, fill]).reshape(E_PAD // GROUP, GROUP)

    table = jnp.zeros((NT, D), jnp.float32).at[:N_NODES, :x.shape[1]].set(x)
    zeros_nt = jnp.zeros((NT, D), jnp.float32)

    e48 = jnp.pad(edge_attr.reshape(N_EDGES // 8, 48),
                  ((0, ER - N_EDGES // 8), (0, 0)))

    def edge_w(name, hs_w, e_w):
        (w1, b1), (w2, b2), *tail = params[name]
        wa = _blk(_pad_rows(w1[:hs_w], D))
        we = _blk(w1[hs_w:hs_w + e_w])
        wc = _blk(_pad_rows(w1[hs_w + e_w:], D))
        out = [wa, we, wc, _brow(b1), _blk(w2), _brow(b2)]
        if tail:
            (w3, b3), = tail
            out += [_blk(_pad_cols(w3, D)), _brow(b3)]
        return out

    def node_w(name, hs_w):
        (w1, b1), (w2, b2), *tail = params[name]
        wa = _blk(_pad_rows(w1[:hs_w], D))
        wb = _blk(_pad_rows(w1[hs_w:], D))
        out = [wa, wb, _brow(b1), _blk(w2), _brow(b2)]
        if tail:
            (w3, b3), = tail
            out += [_blk(_pad_cols(w3, D)), _brow(b3)]
        return out

    h_widths = {"1": 12, "2": 16, "3": 16}
    e_widths = {"1": 6, "2": 16, "3": 16}

    e = e48
    for li in ("1", "2", "3"):
        hs, hd = _gather_pallas(table, src2, dst2)
        e = _edge_mlp_pallas(hs.reshape(ER, 128), hd.reshape(ER, 128), e,
                             *edge_w("e" + li, h_widths[li], e_widths[li]))
        if li == "3":
            a0, a1, e_pack = _scatter_pallas(hs, e.reshape(E_PAD, D), dst2,
                                             zeros_nt, pack_e=True)
        else:
            a0, a1 = _scatter_pallas(hs, e.reshape(E_PAD, D), dst2, zeros_nt)
        table8 = _node_mlp_pallas(a0.reshape(NT // 8, 128),
                                  a1.reshape(NT // 8, 128),
                                  *node_w("n" + li, h_widths[li]))
        table = table8.reshape(NT, D)

    h_out = table[:N_NODES, :3]
    e_out = e_pack.reshape(3, E_PAD)[:, :N_EDGES].T
    return (h_out, e_out)
